# baseline (speedup 1.0000x reference)
import functools

import jax
import jax.numpy as jnp
from jax import lax
from jax.experimental import pallas as pl
from jax.experimental.pallas import tpu as pltpu
from jax.experimental.pallas import tpu_sc as plsc

_SEQ = 256
_NG = 5
_NH = 32
_LANES = 16
_SUBCORES = 16


@jax.jit
def _fg_bias_sc(bias_matrix, g):
    mesh = plsc.VectorSubcoreMesh(core_axis_name="c", subcore_axis_name="s")

    @functools.partial(
        pl.kernel,
        mesh=mesh,
        out_type=jax.ShapeDtypeStruct((_NH, _SEQ, _SEQ), jnp.float32),
        compiler_params=pltpu.CompilerParams(
            use_tc_tiling_on_sc=False, needs_layout_passes=False
        ),
        scratch_types=[
            pltpu.VMEM((_NH, _NG, _NG), jnp.float32),
            pltpu.VMEM((_SEQ,), jnp.int32),
            pltpu.VMEM((8, _SEQ), jnp.float32),
            pltpu.VMEM((4, 64), jnp.int32),
            pltpu.VMEM((_SEQ, _SEQ), jnp.float32),
            pltpu.VMEM_SHARED((_SUBCORES * 8, _SEQ), jnp.float32),
            [pltpu.SemaphoreType.DMA] * 4,
            [pltpu.SemaphoreType.DMA] * 4,
        ],
    )
    def k(b_hbm, g_hbm, out_hbm, b_v, g_v, tmp_v, idx_v, out_v, tbl_sh, gsem, wsem):
        c = lax.axis_index("c")
        s = lax.axis_index("s")
        h = c * _SUBCORES + s
        pltpu.sync_copy(g_hbm, g_v)
        pltpu.sync_copy(b_hbm, b_v)

        h_splat = jnp.full((_LANES,), h, dtype=jnp.int32)
        for a in range(_NG):
            a_splat = jnp.full((_LANES,), a, dtype=jnp.int32)
            for ch in range(_SEQ // _LANES):
                gj = g_v[pl.ds(ch * _LANES, _LANES)]
                tmp_v[a, pl.ds(ch * _LANES, _LANES)] = plsc.load_gather(
                    b_v, [h_splat, a_splat, gj]
                )

        pltpu.sync_copy(tmp_v, tbl_sh.at[pl.ds(s * 8, 8)])

        for ch in range(_SEQ // _LANES):
            gj = g_v[pl.ds(ch * _LANES, _LANES)]
            r, off = divmod(ch * _LANES, 64)
            idx_v[r, pl.ds(off, _LANES)] = gj + s * 8

        plsc.subcore_barrier()

        ob = out_hbm.at[h]
        gcp = [
            pltpu.async_copy(
                tbl_sh.at[idx_v.at[t]], out_v.at[pl.ds(t * 64, 64)], gsem[t]
            )
            for t in range(4)
        ]
        wcp = []
        for t in range(4):
            gcp[t].wait()
            wcp.append(
                pltpu.async_copy(
                    out_v.at[pl.ds(t * 64, 64)], ob.at[pl.ds(t * 64, 64)], wsem[t]
                )
            )
        for c2 in wcp:
            c2.wait()

    return k(bias_matrix, g)


def kernel(bias_matrix, group_assignment):
    return _fg_bias_sc(bias_matrix, group_assignment)


# speedup vs baseline: 1.3694x; 1.3694x over previous
import functools

import jax
import jax.numpy as jnp
from jax import lax
from jax.experimental import pallas as pl
from jax.experimental.pallas import tpu as pltpu
from jax.experimental.pallas import tpu_sc as plsc

_SEQ = 256
_NG = 5
_NH = 32
_LANES = 16
_SUBCORES = 16


@jax.jit
def _fg_bias_sc(bias_matrix, g):
    mesh = plsc.VectorSubcoreMesh(core_axis_name="c", subcore_axis_name="s")

    @functools.partial(
        pl.kernel,
        mesh=mesh,
        out_type=jax.ShapeDtypeStruct((_NH, 512, 128), jnp.float32),
        compiler_params=pltpu.CompilerParams(
            use_tc_tiling_on_sc=False, needs_layout_passes=False
        ),
        scratch_types=[
            pltpu.VMEM((_NH, _NG, _NG), jnp.float32),
            pltpu.VMEM((_SEQ + _LANES,), jnp.int32),
            pltpu.VMEM((16, 128), jnp.float32),
            pltpu.VMEM((4, 128), jnp.int32),
            pltpu.VMEM((512, 128), jnp.float32),
            pltpu.VMEM_SHARED((_SUBCORES * 16, 128), jnp.float32),
            [pltpu.SemaphoreType.DMA] * 4,
            [pltpu.SemaphoreType.DMA] * 4,
        ],
    )
    def k(b_hbm, g_hbm, out_hbm, b_v, g_v, tmp_v, idx_v, out_v, tbl_sh, gsem, wsem):
        c = lax.axis_index("c")
        s = lax.axis_index("s")
        h = c * _SUBCORES + s
        pltpu.sync_copy(g_hbm, g_v.at[pl.ds(0, _SEQ)])
        pltpu.sync_copy(b_hbm, b_v)

        h_splat = jnp.full((_LANES,), h, dtype=jnp.int32)
        for a in range(_NG):
            a_splat = jnp.full((_LANES,), a, dtype=jnp.int32)
            for ch in range(_SEQ // _LANES):
                gj = g_v[pl.ds(ch * _LANES, _LANES)]
                jt, jc = divmod(ch * _LANES, 128)
                tmp_v[a * 2 + jt, pl.ds(jc, _LANES)] = plsc.load_gather(
                    b_v, [h_splat, a_splat, gj]
                )

        pltpu.sync_copy(tmp_v, tbl_sh.at[pl.ds(s * 16, 16)])

        lane = jnp.arange(_LANES, dtype=jnp.int32)
        perm = lane % 8
        jtvec = lane // 8
        for it in range(32):
            gchunk = g_v[pl.ds(it * 8, _LANES)]
            gdup = lax.gather(
                gchunk,
                perm[:, None],
                lax.GatherDimensionNumbers(
                    offset_dims=(),
                    collapsed_slice_dims=(0,),
                    start_index_map=(0,),
                ),
                (1,),
                mode=lax.GatherScatterMode.PROMISE_IN_BOUNDS,
            )
            vals = gdup * 2 + jtvec + s * 16
            r, off = divmod(it * _LANES, 128)
            idx_v[r, pl.ds(off, _LANES)] = vals

        plsc.subcore_barrier()

        ob = out_hbm.at[h]
        gcp = [
            pltpu.async_copy(
                tbl_sh.at[idx_v.at[t]], out_v.at[pl.ds(t * 128, 128)], gsem[t]
            )
            for t in range(4)
        ]
        wcp = []
        for t in range(4):
            gcp[t].wait()
            wcp.append(
                pltpu.async_copy(
                    out_v.at[pl.ds(t * 128, 128)], ob.at[pl.ds(t * 128, 128)], wsem[t]
                )
            )
        for c2 in wcp:
            c2.wait()

    out = k(bias_matrix, g)
    return (
        out.reshape(_NH, 32, 2, 8, 128)
        .transpose(0, 1, 3, 2, 4)
        .reshape(_NH, _SEQ, _SEQ)
    )


def kernel(bias_matrix, group_assignment):
    return _fg_bias_sc(bias_matrix, group_assignment)


# speedup vs baseline: 1.3816x; 1.0089x over previous
import functools

import jax
import jax.numpy as jnp
from jax import lax
from jax.experimental import pallas as pl
from jax.experimental.pallas import tpu as pltpu
from jax.experimental.pallas import tpu_sc as plsc

_SEQ = 256
_NG = 5
_NH = 32
_LANES = 16
_SUBCORES = 16


@jax.jit
def _fg_bias_sc(bias_matrix, g):
    mesh = plsc.VectorSubcoreMesh(core_axis_name="c", subcore_axis_name="s")

    @functools.partial(
        pl.kernel,
        mesh=mesh,
        out_type=jax.ShapeDtypeStruct((_NH, 512, 128), jnp.float32),
        compiler_params=pltpu.CompilerParams(
            use_tc_tiling_on_sc=False, needs_layout_passes=False
        ),
        scratch_types=[
            pltpu.VMEM((_NH, _NG, _NG), jnp.float32),
            pltpu.VMEM((_SEQ + _LANES,), jnp.int32),
            pltpu.VMEM((16, 128), jnp.float32),
            pltpu.VMEM((4, 128), jnp.int32),
            pltpu.VMEM((512, 128), jnp.float32),
            pltpu.VMEM_SHARED((_SUBCORES * 16, 128), jnp.float32),
            [pltpu.SemaphoreType.DMA] * 4,
            [pltpu.SemaphoreType.DMA] * 4,
        ],
    )
    def k(b_hbm, g_hbm, out_hbm, b_v, g_v, tmp_v, idx_v, out_v, tbl_sh, gsem, wsem):
        c = lax.axis_index("c")
        s = lax.axis_index("s")
        h = c * _SUBCORES + s
        pltpu.sync_copy(g_hbm, g_v.at[pl.ds(0, _SEQ)])
        pltpu.sync_copy(b_hbm, b_v)

        h_splat = jnp.full((_LANES,), h, dtype=jnp.int32)
        for a in range(_NG):
            a_splat = jnp.full((_LANES,), a, dtype=jnp.int32)
            for ch in range(_SEQ // _LANES):
                gj = g_v[pl.ds(ch * _LANES, _LANES)]
                jt, jc = divmod(ch * _LANES, 128)
                tmp_v[a * 2 + jt, pl.ds(jc, _LANES)] = plsc.load_gather(
                    b_v, [h_splat, a_splat, gj]
                )

        pltpu.sync_copy(tmp_v, tbl_sh.at[pl.ds(s * 16, 16)])

        lane = jnp.arange(_LANES, dtype=jnp.int32)
        perm = lane % 8
        jtvec = lane // 8
        for it in range(32):
            gchunk = g_v[pl.ds(it * 8, _LANES)]
            gdup = lax.gather(
                gchunk,
                perm[:, None],
                lax.GatherDimensionNumbers(
                    offset_dims=(),
                    collapsed_slice_dims=(0,),
                    start_index_map=(0,),
                ),
                (1,),
                mode=lax.GatherScatterMode.PROMISE_IN_BOUNDS,
            )
            vals = gdup * 2 + jtvec + s * 16
            r, off = divmod(it * _LANES, 128)
            idx_v[r, pl.ds(off, _LANES)] = vals


        ob = out_hbm.at[h]
        gcp = [
            pltpu.async_copy(
                tbl_sh.at[idx_v.at[t]], out_v.at[pl.ds(t * 128, 128)], gsem[t]
            )
            for t in range(4)
        ]
        wcp = []
        for t in range(4):
            gcp[t].wait()
            wcp.append(
                pltpu.async_copy(
                    out_v.at[pl.ds(t * 128, 128)], ob.at[pl.ds(t * 128, 128)], wsem[t]
                )
            )
        for c2 in wcp:
            c2.wait()

    out = k(bias_matrix, g)
    return (
        out.reshape(_NH, 32, 2, 8, 128)
        .transpose(0, 1, 3, 2, 4)
        .reshape(_NH, _SEQ, _SEQ)
    )


def kernel(bias_matrix, group_assignment):
    return _fg_bias_sc(bias_matrix, group_assignment)


# speedup vs baseline: 1.3865x; 1.0036x over previous
import functools

import jax
import jax.numpy as jnp
from jax import lax
from jax.experimental import pallas as pl
from jax.experimental.pallas import tpu as pltpu
from jax.experimental.pallas import tpu_sc as plsc

_SEQ = 256
_NG = 5
_NH = 32
_LANES = 16
_SUBCORES = 16


@jax.jit
def _fg_bias_sc(bias_matrix, g):
    mesh = plsc.VectorSubcoreMesh(core_axis_name="c", subcore_axis_name="s")

    @functools.partial(
        pl.kernel,
        mesh=mesh,
        out_type=jax.ShapeDtypeStruct((_NH, 512, 128), jnp.float32),
        compiler_params=pltpu.CompilerParams(
            use_tc_tiling_on_sc=False, needs_layout_passes=False, skip_device_barrier=True
        ),
        scratch_types=[
            pltpu.VMEM((_NH, _NG, _NG), jnp.float32),
            pltpu.VMEM((_SEQ + _LANES,), jnp.int32),
            pltpu.VMEM((16, 128), jnp.float32),
            pltpu.VMEM((4, 128), jnp.int32),
            pltpu.VMEM((512, 128), jnp.float32),
            pltpu.VMEM_SHARED((_SUBCORES * 16, 128), jnp.float32),
            [pltpu.SemaphoreType.DMA] * 4,
            [pltpu.SemaphoreType.DMA] * 4,
        ],
    )
    def k(b_hbm, g_hbm, out_hbm, b_v, g_v, tmp_v, idx_v, out_v, tbl_sh, gsem, wsem):
        c = lax.axis_index("c")
        s = lax.axis_index("s")
        h = c * _SUBCORES + s
        pltpu.sync_copy(g_hbm, g_v.at[pl.ds(0, _SEQ)])
        pltpu.sync_copy(b_hbm, b_v)

        h_splat = jnp.full((_LANES,), h, dtype=jnp.int32)
        for a in range(_NG):
            a_splat = jnp.full((_LANES,), a, dtype=jnp.int32)
            for ch in range(_SEQ // _LANES):
                gj = g_v[pl.ds(ch * _LANES, _LANES)]
                jt, jc = divmod(ch * _LANES, 128)
                tmp_v[a * 2 + jt, pl.ds(jc, _LANES)] = plsc.load_gather(
                    b_v, [h_splat, a_splat, gj]
                )

        pltpu.sync_copy(tmp_v, tbl_sh.at[pl.ds(s * 16, 16)])

        lane = jnp.arange(_LANES, dtype=jnp.int32)
        perm = lane % 8
        jtvec = lane // 8
        for it in range(32):
            gchunk = g_v[pl.ds(it * 8, _LANES)]
            gdup = lax.gather(
                gchunk,
                perm[:, None],
                lax.GatherDimensionNumbers(
                    offset_dims=(),
                    collapsed_slice_dims=(0,),
                    start_index_map=(0,),
                ),
                (1,),
                mode=lax.GatherScatterMode.PROMISE_IN_BOUNDS,
            )
            vals = gdup * 2 + jtvec + s * 16
            r, off = divmod(it * _LANES, 128)
            idx_v[r, pl.ds(off, _LANES)] = vals


        ob = out_hbm.at[h]
        gcp = [
            pltpu.async_copy(
                tbl_sh.at[idx_v.at[t]], out_v.at[pl.ds(t * 128, 128)], gsem[t]
            )
            for t in range(4)
        ]
        wcp = []
        for t in range(4):
            gcp[t].wait()
            wcp.append(
                pltpu.async_copy(
                    out_v.at[pl.ds(t * 128, 128)], ob.at[pl.ds(t * 128, 128)], wsem[t]
                )
            )
        for c2 in wcp:
            c2.wait()

    out = k(bias_matrix, g)
    return (
        out.reshape(_NH, 32, 2, 8, 128)
        .transpose(0, 1, 3, 2, 4)
        .reshape(_NH, _SEQ, _SEQ)
    )


def kernel(bias_matrix, group_assignment):
    return _fg_bias_sc(bias_matrix, group_assignment)
